# baseline (speedup 1.0000x reference)
import jax
import jax.numpy as jnp
from jax.experimental import pallas as pl
from jax.experimental.pallas import tpu as pltpu
from jax.experimental.pallas import tpu_sc as plsc

_EMBED_DIM = 128
_GROUP = 8


def _sc_gather(table, idx):
    n_batch, n_tok = idx.shape
    vector_mesh = plsc.VectorSubcoreMesh(
        core_axis_name="core", subcore_axis_name="subcore"
    )
    n_workers = 32
    rows_per_worker = n_batch // n_workers
    n_groups = rows_per_worker // _GROUP
    n_pairs = n_groups // 2

    @pl.kernel(
        out_type=jax.ShapeDtypeStruct((n_batch, n_tok, _EMBED_DIM), table.dtype),
        mesh=vector_mesh,
        scratch_types=[
            pltpu.VMEM((_GROUP, n_tok), jnp.int32),
            pltpu.VMEM((_GROUP, n_tok), jnp.int32),
            pltpu.VMEM((_GROUP, n_tok, _EMBED_DIM), table.dtype),
            pltpu.VMEM((_GROUP, n_tok, _EMBED_DIM), table.dtype),
            pltpu.SemaphoreType.DMA,
            pltpu.SemaphoreType.DMA,
            pltpu.SemaphoreType.DMA,
        ],
    )
    def gather_kernel(table_hbm, idx_hbm, out_hbm, idx_v0, idx_v1, rows_v0,
                      rows_v1, isem, gsem, wsem):
        wid = jax.lax.axis_index("subcore") * 2 + jax.lax.axis_index("core")
        base = wid * rows_per_worker
        idx_v = (idx_v0, idx_v1)
        rows_v = (rows_v0, rows_v1)

        def idx_load(g, b):
            return pltpu.async_copy(
                idx_hbm.at[pl.ds(base + g * _GROUP, _GROUP)], idx_v[b], isem
            )

        def idx_wait(g, b):
            pltpu.make_async_copy(
                idx_hbm.at[pl.ds(base + g * _GROUP, _GROUP)], idx_v[b], isem
            ).wait()

        def fire_gathers(b):
            for j in range(_GROUP):
                pltpu.async_copy(table_hbm.at[idx_v[b].at[j]],
                                 rows_v[b].at[j], gsem)

        def wait_gathers(b):
            for j in range(_GROUP):
                pltpu.make_async_copy(table_hbm.at[idx_v[b].at[j]],
                                      rows_v[b].at[j], gsem).wait()

        def fire_writebacks(g, b):
            r0 = base + g * _GROUP
            for j in range(_GROUP):
                pltpu.async_copy(rows_v[b].at[j], out_hbm.at[r0 + j], wsem)

        def drain_writebacks(g, b):
            r0 = base + g * _GROUP
            for j in range(_GROUP):
                pltpu.make_async_copy(rows_v[b].at[j], out_hbm.at[r0 + j],
                                      wsem).wait()

        idx_load(0, 0).wait()
        fire_gathers(0)
        idx_load(1, 1)

        @pl.loop(0, n_pairs)
        def _(t):
            g0 = 2 * t
            g1 = g0 + 1
            @pl.when(t > 0)
            def _():
                drain_writebacks(g0 - 1, 1)

            idx_wait(g1, 1)
            fire_gathers(1)

            wait_gathers(0)
            fire_writebacks(g0, 0)

            @pl.when(t < n_pairs - 1)
            def _():
                idx_load(g0 + 2, 0)
                drain_writebacks(g0, 0)
                idx_wait(g0 + 2, 0)
                fire_gathers(0)

            wait_gathers(1)
            fire_writebacks(g1, 1)

            @pl.when(t < n_pairs - 1)
            def _():
                idx_load(g1 + 2, 1)

        drain_writebacks(n_groups - 2, 0)
        drain_writebacks(n_groups - 1, 1)

    return gather_kernel(table, idx)


_CHUNKS = 4


def kernel(x, table):
    idx = x[0]
    nb = idx.shape[0] // _CHUNKS
    outs = [
        jax.lax.optimization_barrier(
            _sc_gather(table, idx[c * nb:(c + 1) * nb]))
        for c in range(_CHUNKS)
    ]
    return jnp.concatenate(outs, axis=0)


# speedup vs baseline: 1.0891x; 1.0891x over previous
import jax
import jax.numpy as jnp
from jax.experimental import pallas as pl
from jax.experimental.pallas import tpu as pltpu
from jax.experimental.pallas import tpu_sc as plsc

_EMBED_DIM = 128
_GROUP = 8


def _sc_gather(table, idx):
    n_batch, n_tok = idx.shape
    vector_mesh = plsc.VectorSubcoreMesh(
        core_axis_name="core", subcore_axis_name="subcore"
    )
    n_workers = 32
    rows_per_worker = n_batch // n_workers
    n_groups = rows_per_worker // _GROUP
    n_pairs = n_groups // 2

    @pl.kernel(
        out_type=jax.ShapeDtypeStruct((n_batch, n_tok, _EMBED_DIM), table.dtype),
        mesh=vector_mesh,
        scratch_types=[
            pltpu.VMEM((_GROUP, n_tok), jnp.int32),
            pltpu.VMEM((_GROUP, n_tok), jnp.int32),
            pltpu.VMEM((_GROUP, n_tok, _EMBED_DIM), table.dtype),
            pltpu.VMEM((_GROUP, n_tok, _EMBED_DIM), table.dtype),
            pltpu.SemaphoreType.DMA,
            pltpu.SemaphoreType.DMA,
            pltpu.SemaphoreType.DMA,
        ],
    )
    def gather_kernel(table_hbm, idx_hbm, out_hbm, idx_v0, idx_v1, rows_v0,
                      rows_v1, isem, gsem, wsem):
        wid = jax.lax.axis_index("subcore") * 2 + jax.lax.axis_index("core")
        base = wid * rows_per_worker
        idx_v = (idx_v0, idx_v1)
        rows_v = (rows_v0, rows_v1)

        def idx_load(g, b):
            return pltpu.async_copy(
                idx_hbm.at[pl.ds(base + g * _GROUP, _GROUP)], idx_v[b], isem
            )

        def idx_wait(g, b):
            pltpu.make_async_copy(
                idx_hbm.at[pl.ds(base + g * _GROUP, _GROUP)], idx_v[b], isem
            ).wait()

        def fire_gathers(b):
            for j in range(_GROUP):
                pltpu.async_copy(table_hbm.at[idx_v[b].at[j]],
                                 rows_v[b].at[j], gsem)

        def wait_gathers(b):
            for j in range(_GROUP):
                pltpu.make_async_copy(table_hbm.at[idx_v[b].at[j]],
                                      rows_v[b].at[j], gsem).wait()

        def fire_writebacks(g, b):
            r0 = base + g * _GROUP
            for j in range(_GROUP):
                pltpu.async_copy(rows_v[b].at[j], out_hbm.at[r0 + j], wsem)

        def drain_writebacks(g, b):
            r0 = base + g * _GROUP
            for j in range(_GROUP):
                pltpu.make_async_copy(rows_v[b].at[j], out_hbm.at[r0 + j],
                                      wsem).wait()

        idx_load(0, 0).wait()
        fire_gathers(0)
        idx_load(1, 1)

        @pl.loop(0, n_pairs)
        def _(t):
            g0 = 2 * t
            g1 = g0 + 1
            @pl.when(t > 0)
            def _():
                drain_writebacks(g0 - 1, 1)

            idx_wait(g1, 1)
            fire_gathers(1)

            wait_gathers(0)
            fire_writebacks(g0, 0)

            @pl.when(t < n_pairs - 1)
            def _():
                idx_load(g0 + 2, 0)
                drain_writebacks(g0, 0)
                idx_wait(g0 + 2, 0)
                fire_gathers(0)

            wait_gathers(1)
            fire_writebacks(g1, 1)

            @pl.when(t < n_pairs - 1)
            def _():
                idx_load(g1 + 2, 1)

        drain_writebacks(n_groups - 2, 0)
        drain_writebacks(n_groups - 1, 1)

    return gather_kernel(table, idx)


def kernel(x, table):
    idx = x[0]
    half = idx.shape[0] // 2
    a = _sc_gather(table, idx[:half])
    b = _sc_gather(table, idx[half:])
    out = jnp.zeros((idx.shape[0], idx.shape[1], _EMBED_DIM), table.dtype)
    out = jax.lax.dynamic_update_slice(out, a, (0, 0, 0))
    out = jax.lax.dynamic_update_slice(out, b, (half, 0, 0))
    return out


# speedup vs baseline: 1.8223x; 1.6732x over previous
import jax
import jax.numpy as jnp
from jax.experimental import pallas as pl
from jax.experimental.pallas import tpu as pltpu
from jax.experimental.pallas import tpu_sc as plsc

_EMBED_DIM = 128
_GROUP = 8


def _sc_gather(table, idx):
    n_batch, n_tok = idx.shape
    vector_mesh = plsc.VectorSubcoreMesh(
        core_axis_name="core", subcore_axis_name="subcore"
    )
    n_workers = 32
    rows_per_worker = n_batch // n_workers
    n_groups = rows_per_worker // _GROUP
    n_pairs = n_groups // 2

    @pl.kernel(
        out_type=jax.ShapeDtypeStruct((n_batch, n_tok, _EMBED_DIM), table.dtype),
        mesh=vector_mesh,
        scratch_types=[
            pltpu.VMEM((_GROUP, n_tok), jnp.int32),
            pltpu.VMEM((_GROUP, n_tok), jnp.int32),
            pltpu.VMEM((_GROUP, n_tok, _EMBED_DIM), table.dtype),
            pltpu.VMEM((_GROUP, n_tok, _EMBED_DIM), table.dtype),
            pltpu.SemaphoreType.DMA,
            pltpu.SemaphoreType.DMA,
            pltpu.SemaphoreType.DMA,
        ],
    )
    def gather_kernel(table_hbm, idx_hbm, out_hbm, idx_v0, idx_v1, rows_v0,
                      rows_v1, isem, gsem, wsem):
        wid = jax.lax.axis_index("subcore") * 2 + jax.lax.axis_index("core")
        base = wid * rows_per_worker
        idx_v = (idx_v0, idx_v1)
        rows_v = (rows_v0, rows_v1)

        def idx_load(g, b):
            return pltpu.async_copy(
                idx_hbm.at[pl.ds(base + g * _GROUP, _GROUP)], idx_v[b], isem
            )

        def idx_wait(g, b):
            pltpu.make_async_copy(
                idx_hbm.at[pl.ds(base + g * _GROUP, _GROUP)], idx_v[b], isem
            ).wait()

        def fire_gathers(b):
            for j in range(_GROUP):
                pltpu.async_copy(table_hbm.at[idx_v[b].at[j]],
                                 rows_v[b].at[j], gsem)

        def wait_gathers(b):
            for j in range(_GROUP):
                pltpu.make_async_copy(table_hbm.at[idx_v[b].at[j]],
                                      rows_v[b].at[j], gsem).wait()

        def fire_writebacks(g, b):
            r0 = base + g * _GROUP
            for j in range(_GROUP):
                pltpu.async_copy(rows_v[b].at[j], out_hbm.at[r0 + j], wsem)

        def drain_writebacks(g, b):
            r0 = base + g * _GROUP
            for j in range(_GROUP):
                pltpu.make_async_copy(rows_v[b].at[j], out_hbm.at[r0 + j],
                                      wsem).wait()

        idx_load(0, 0).wait()
        fire_gathers(0)
        idx_load(1, 1)

        @pl.loop(0, n_pairs)
        def _(t):
            g0 = 2 * t
            g1 = g0 + 1
            @pl.when(t > 0)
            def _():
                drain_writebacks(g0 - 1, 1)

            idx_wait(g1, 1)
            fire_gathers(1)

            wait_gathers(0)
            fire_writebacks(g0, 0)

            @pl.when(t < n_pairs - 1)
            def _():
                idx_load(g0 + 2, 0)
                drain_writebacks(g0, 0)
                idx_wait(g0 + 2, 0)
                fire_gathers(0)

            wait_gathers(1)
            fire_writebacks(g1, 1)

            @pl.when(t < n_pairs - 1)
            def _():
                idx_load(g1 + 2, 1)

        drain_writebacks(n_groups - 2, 0)
        drain_writebacks(n_groups - 1, 1)

    return gather_kernel(table, idx)


def kernel(x, table):
    return _sc_gather(table, x[0])
